# Initial kernel scaffold; baseline (speedup 1.0000x reference)
import jax
import jax.numpy as jnp
from jax.experimental import pallas as pl


def kernel(x, params):
    raise NotImplementedError("write your pallas kernel here")


# speedup vs baseline: 4.3467x; 4.3467x over previous
import functools

import numpy as np
import jax
import jax.numpy as jnp
from jax.experimental import pallas as pl
from jax.experimental.pallas import tpu as pltpu
from jax.experimental.pallas import tpu_sc as plsc

_XCONV_CFG = [
    {'K': 8,  'D': 1, 'P': -1,   'C': 64},
    {'K': 12, 'D': 2, 'P': 768,  'C': 128},
    {'K': 16, 'D': 2, 'P': 384,  'C': 256},
    {'K': 16, 'D': 4, 'P': 128,  'C': 512},
]
_XDCONV_CFG = [
    {'K': 16, 'D': 4, 'pts_layer_idx': 3, 'qrs_layer_idx': 2},
    {'K': 16, 'D': 2, 'pts_layer_idx': 2, 'qrs_layer_idx': 1},
    {'K': 12, 'D': 2, 'pts_layer_idx': 1, 'qrs_layer_idx': 0},
]
_NFC = 3


def _elu(x):
    return jnp.where(x > 0, x, jnp.exp(x) - 1.0)


def _knn_body(KD1, P, rep_ref, ptst_ref, out_ref):
    rep = rep_ref[0]
    d2 = None
    for d in range(3):
        diff = rep[:, d:d + 1] - ptst_ref[0, d:d + 1, :]
        sq = diff * diff
        d2 = sq if d2 is None else d2 + sq
    Rb = rep.shape[0]
    iota = jax.lax.broadcasted_iota(jnp.int32, (Rb, P), 1).astype(jnp.float32)
    cur = d2
    for i in range(KD1):
        m = jnp.min(cur, axis=1, keepdims=True)
        idxf = jnp.min(jnp.where(cur == m, iota, jnp.float32(P)),
                       axis=1, keepdims=True)
        out_ref[0, :, i:i + 1] = idxf.astype(jnp.int32)
        cur = jnp.where(iota == idxf, jnp.float32(jnp.inf), cur)


def _knn(rep, pts, KD1):
    B, R, _ = rep.shape
    P = pts.shape[1]
    ptst = jnp.transpose(pts, (0, 2, 1))
    Rb = 128
    return pl.pallas_call(
        functools.partial(_knn_body, KD1, P),
        grid=(B, R // Rb),
        in_specs=[
            pl.BlockSpec((1, Rb, 3), lambda b, r: (b, r, 0)),
            pl.BlockSpec((1, 3, P), lambda b, r: (b, 0, 0)),
        ],
        out_specs=pl.BlockSpec((1, Rb, KD1), lambda b, r: (b, r, 0)),
        out_shape=jax.ShapeDtypeStruct((B, R, KD1), jnp.int32),
    )(rep, ptst)


def _sc_gather(data2d, idx_flat):
    N = idx_flat.shape[0]
    C = data2d.shape[1]
    window = 512
    while window > 8 and (N % window != 0 or window * C * 4 > 131072):
        window //= 2
    mesh = plsc.VectorSubcoreMesh(core_axis_name="c", subcore_axis_name="s")
    idx2 = idx_flat.reshape(N // window, window)

    @functools.partial(pl.kernel,
                       out_type=jax.ShapeDtypeStruct((N, C), data2d.dtype),
                       mesh=mesh)
    def _gather_kernel(x_hbm, i_hbm, o_hbm):
        def body(i_vmem, o_vmem):
            pltpu.sync_copy(x_hbm.at[i_vmem.at[0]], o_vmem)

        pltpu.emit_pipeline(
            body,
            grid=(N // window,),
            in_specs=[pl.BlockSpec((1, window), lambda i: (i, 0))],
            out_specs=[pl.BlockSpec((window, C), lambda i: (i, 0))],
            core_axis_name=("c", "s"),
            dimension_semantics=(pltpu.PARALLEL,),
        )(i_hbm, o_hbm)

    return _gather_kernel(data2d, idx2)


def _mlp_body(nlayers, *refs):
    x_ref = refs[0]
    o_ref = refs[-1]
    h = x_ref[...]
    for i in range(nlayers):
        W = refs[1 + 2 * i][...]
        b = refs[2 + 2 * i][...]
        h = _elu(jnp.dot(h, W, preferred_element_type=jnp.float32) + b)
    o_ref[...] = h


def _mlp(x2d, layers):
    M = x2d.shape[0]
    Cout = layers[-1][0].shape[1]
    args = [x2d]
    for W, b in layers:
        args += [W, b.reshape(1, -1)]
    return pl.pallas_call(
        functools.partial(_mlp_body, len(layers)),
        out_shape=jax.ShapeDtypeStruct((M, Cout), jnp.float32),
    )(*args)


def _fold_body(dm, dw_ref, pwt_ref, o_ref):
    dwk = dw_ref[0]
    acc = None
    for m in range(dm):
        t = dwk[:, m:m + 1] * pwt_ref[m]
        acc = t if acc is None else acc + t
    o_ref[0] = acc


def _fold(dwaug, pw, Ctot, dm, Cout):
    K1 = dwaug.shape[0]
    pwt = pw.reshape(Ctot, dm, Cout).transpose(1, 0, 2)
    cb = Cout
    while dm * Ctot * cb * 4 > 16 * 1024 * 1024:
        cb //= 2
    return pl.pallas_call(
        functools.partial(_fold_body, dm),
        grid=(Cout // cb, K1),
        in_specs=[
            pl.BlockSpec((1, Ctot, dm), lambda ci, k: (k, 0, 0)),
            pl.BlockSpec((dm, Ctot, cb), lambda ci, k: (0, 0, ci)),
        ],
        out_specs=pl.BlockSpec((1, Ctot, cb), lambda ci, k: (k, 0, ci)),
        out_shape=jax.ShapeDtypeStruct((K1, Ctot, Cout), jnp.float32),
    )(dwaug, pwt)


def _xconv_body(K, Cmid, has_fts, dm1, *refs):
    rep_ref, pts_ref = refs[0], refs[1]
    refs = refs[2:]
    if has_fts:
        fts_ref, refs = refs[0], refs[1:]
    if dm1:
        (Wd1, bd1, Wd2, bd2, Wx0, bx0, Wx1, bx1, Wx2, bx2, dwr, dwb, pw,
         pwb, o_ref) = refs
    else:
        (Wd1, bd1, Wd2, bd2, Wx0, bx0, Wx1, bx1, Wx2, bx2, Waug, pwb,
         o_ref) = refs

    rep = rep_ref[0]
    ploc = [pts_ref[0, k] - rep for k in range(K)]

    f2 = []
    for k in range(K):
        acc = None
        for d in range(3):
            t = ploc[k][:, d:d + 1] * Wd1[d:d + 1, :]
            acc = t if acc is None else acc + t
        h = _elu(acc + bd1[...])
        h = _elu(jnp.dot(h, Wd2[...],
                               preferred_element_type=jnp.float32) + bd2[...])
        f2.append(h)

    Xacc = None
    for k in range(K):
        for d in range(3):
            t = ploc[k][:, d:d + 1] * Wx0[k * 3 + d:k * 3 + d + 1, :]
            Xacc = t if Xacc is None else Xacc + t
    X = _elu(Xacc + bx0[...])
    X = _elu(jnp.dot(X, Wx1[...],
                           preferred_element_type=jnp.float32) + bx1[...])
    X = jnp.dot(X, Wx2[...], preferred_element_type=jnp.float32) + bx2[...]

    if has_fts:
        fts = [fts_ref[0, j] for j in range(K)]

    if dm1:
        dw2f = None
        dw2r = None
        for k in range(K):
            accf = None
            accr = None
            for j in range(K):
                c = X[:, k * K + j:k * K + j + 1]
                t = c * f2[j]
                accf = t if accf is None else accf + t
                if has_fts:
                    t = c * fts[j]
                    accr = t if accr is None else accr + t
            tf = accf * dwr[k:k + 1, :Cmid]
            dw2f = tf if dw2f is None else dw2f + tf
            if has_fts:
                tr = accr * dwr[k:k + 1, Cmid:]
                dw2r = tr if dw2r is None else dw2r + tr
        out = jnp.dot(dw2f + dwb[:, :Cmid], pw[:Cmid, :],
                      preferred_element_type=jnp.float32)
        if has_fts:
            out = out + jnp.dot(dw2r + dwb[:, Cmid:], pw[Cmid:, :],
                                preferred_element_type=jnp.float32)
        o_ref[0] = _elu(out + pwb[...])
        return

    out = None
    for k in range(K):
        accf = None
        for j in range(K):
            c = X[:, k * K + j:k * K + j + 1]
            t = c * f2[j]
            accf = t if accf is None else accf + t
        term = jnp.dot(accf, Waug[k, :Cmid, :],
                       preferred_element_type=jnp.float32)
        if has_fts:
            accr = None
            for j in range(K):
                c = X[:, k * K + j:k * K + j + 1]
                t = c * fts[j]
                accr = t if accr is None else accr + t
            term = term + jnp.dot(accr, Waug[k, Cmid:, :],
                                  preferred_element_type=jnp.float32)
        out = term if out is None else out + term

    bias2 = jnp.sum(Waug[K], axis=0, keepdims=True) + pwb[...]
    o_ref[0] = _elu(out + bias2)


def _xconv(p, rep16, pts16, fts, K, Cmid, Cin, dm, Cout):
    B, R = rep16.shape[0], rep16.shape[1]
    Ctot = Cmid + Cin
    dm1 = dm == 1

    per_row = K * (Cin + Cmid + 16) * 4
    cap = 6 * 1024 * 1024
    Rb = R
    if R * per_row > cap:
        Rb = next((c for c in (512, 384, 256, 128)
                   if R % c == 0 and c * per_row <= cap), 128)
    args = [rep16, pts16]
    in_specs = [
        pl.BlockSpec((1, Rb, 16), lambda b, r, co: (b, r, 0)),
        pl.BlockSpec((1, K, Rb, 16), lambda b, r, co: (b, 0, r, 0)),
    ]
    if fts is not None:
        args.append(fts)
        in_specs.append(
            pl.BlockSpec((1, K, Rb, Cin), lambda b, r, co: (b, 0, r, 0)))
    KK = K * K
    wspecs = [
        (p['d1']['W'], (3, Cmid)), (p['d1']['b'].reshape(1, -1), (1, Cmid)),
        (p['d2']['W'], (Cmid, Cmid)), (p['d2']['b'].reshape(1, -1), (1, Cmid)),
        (p['x0']['W'], (3 * K, KK)), (p['x0']['b'].reshape(1, -1), (1, KK)),
        (p['x1']['W'], (KK, KK)), (p['x1']['b'].reshape(1, -1), (1, KK)),
        (p['x2']['W'], (KK, KK)), (p['x2']['b'].reshape(1, -1), (1, KK)),
    ]
    cob = Cout
    if not dm1:
        while (K + 1) * Ctot * cob * 4 > 6 * 1024 * 1024:
            cob //= 2
    for arr, shp in wspecs:
        args.append(arr)
        in_specs.append(
            pl.BlockSpec(shp, lambda b, r, co, _n=len(shp): (0,) * _n))
    if dm1:
        args += [p['dw'].reshape(K, Ctot), p['dwb'].reshape(1, Ctot),
                 p['pw'], p['pwb'].reshape(1, -1)]
        in_specs += [
            pl.BlockSpec((K, Ctot), lambda b, r, co: (0, 0)),
            pl.BlockSpec((1, Ctot), lambda b, r, co: (0, 0)),
            pl.BlockSpec((Ctot, Cout), lambda b, r, co: (0, 0)),
            pl.BlockSpec((1, Cout), lambda b, r, co: (0, 0)),
        ]
    else:
        dwaug = jnp.concatenate(
            [p['dw'], p['dwb'].reshape(1, Ctot, dm)], axis=0)
        Waug = _fold(dwaug, p['pw'], Ctot, dm, Cout)
        args += [Waug, p['pwb'].reshape(1, -1)]
        in_specs += [
            pl.BlockSpec((K + 1, Ctot, cob), lambda b, r, co: (0, 0, co)),
            pl.BlockSpec((1, cob), lambda b, r, co: (0, co)),
        ]
    return pl.pallas_call(
        functools.partial(_xconv_body, K, Cmid, fts is not None, dm1),
        grid=(B, R // Rb, Cout // cob),
        in_specs=in_specs,
        out_specs=pl.BlockSpec((1, Rb, cob), lambda b, r, co: (b, r, co)),
        out_shape=jax.ShapeDtypeStruct((B, R, Cout), jnp.float32),
    )(*args)


def _pad16(pts):
    B, P, _ = pts.shape
    return jnp.concatenate(
        [pts, jnp.zeros((B, P, 13), jnp.float32)], axis=-1)


def _gather_neighbors(data3d, nidx, keep=None):
    B, P, C = data3d.shape
    _, R, K = nidx.shape
    Cp = ((C + 127) // 128) * 128
    if Cp != C:
        data3d = jnp.concatenate(
            [data3d, jnp.zeros((B, P, Cp - C), jnp.float32)], axis=-1)
    flat = (nidx + (jnp.arange(B, dtype=jnp.int32) * P)[:, None, None])
    g = _sc_gather(data3d.reshape(B * P, Cp), flat.reshape(B * R * K))
    g = g.reshape(B, R, K, Cp)
    if keep is None:
        keep = C
    if keep != Cp:
        g = g[..., :keep]
    return g.transpose(0, 2, 1, 3)


def _nidx(rep, pts, K, D):
    idx = _knn(rep, pts, K * D + 1)
    return idx[:, :, 1::D][:, :, :K]


def kernel(x, params):
    rng = np.random.default_rng(0)
    B, NPTS, _ = x.shape
    C = [c['C'] for c in _XCONV_CFG]
    xc_meta = [
        (0, C[0] // 2, 4),
        (C[1] // 2, C[1] // 4, C[0] // 4),
        (C[2] // 2, C[2] // 4, C[1] // 4),
        (C[3] // 2, C[3] // 4, C[2] // 4),
    ]

    layer_pts = [x]
    outs = [None]
    prev = x
    prev_out = None
    for i, cfg in enumerate(_XCONV_CFG):
        if cfg['P'] != -1:
            sel = rng.choice(prev.shape[1], cfg['P'], replace=False)
            rep = prev[:, sel, :]
        else:
            rep = prev
        Cin, Cmid, dm = xc_meta[i]
        if i == 0:
            fts_full = None
        else:
            dp = params['dense%d' % i]
            fts_full = _mlp(prev_out.reshape(-1, prev_out.shape[-1]),
                            [(dp['W'], dp['b'])]).reshape(
                                prev.shape[0], prev.shape[1], -1)
        nidx = _nidx(rep, prev, cfg['K'], cfg['D'])
        pts16 = _gather_neighbors(prev, nidx, keep=16)
        ftsg = (None if fts_full is None
                else _gather_neighbors(fts_full, nidx))
        out = _xconv(params['xconv%d' % (i + 1)], _pad16(rep), pts16, ftsg,
                     cfg['K'], Cmid, Cin, dm, cfg['C'])
        layer_pts.append(rep)
        outs.append(out)
        prev = rep
        prev_out = out

    for i, cfg in enumerate(_XDCONV_CFG):
        this_out = outs[cfg['pts_layer_idx'] + 1] if i == 0 else outs[-1]
        rep = layer_pts[cfg['qrs_layer_idx'] + 1]
        rep2 = layer_pts[cfg['pts_layer_idx'] + 1]
        ci = this_out.shape[-1]
        co = C[cfg['qrs_layer_idx']]
        nidx = _nidx(rep, rep2, cfg['K'], cfg['D'])
        pts16 = _gather_neighbors(rep2, nidx, keep=16)
        this_r = _gather_neighbors(this_out, nidx)
        out = _xconv(params['deconv%d' % i], _pad16(rep), pts16, this_r,
                     cfg['K'], ci // 4, ci, 1, co)
        cat = jnp.concatenate([out, outs[cfg['qrs_layer_idx'] + 1]], axis=-1)
        dp = params['ddense%d' % i]
        out = _mlp(cat.reshape(-1, cat.shape[-1]),
                   [(dp['W'], dp['b'])]).reshape(B, -1, co)
        outs.append(out)

    h = outs[-1]
    fc_layers = [(params['fc%d' % i]['W'], params['fc%d' % i]['b'])
                 for i in range(_NFC)]
    out = _mlp(h.reshape(-1, h.shape[-1]), fc_layers)
    return out.reshape(B, NPTS, -1)


# speedup vs baseline: 4.3497x; 1.0007x over previous
import functools

import numpy as np
import jax
import jax.numpy as jnp
from jax.experimental import pallas as pl
from jax.experimental.pallas import tpu as pltpu
from jax.experimental.pallas import tpu_sc as plsc

_XCONV_CFG = [
    {'K': 8,  'D': 1, 'P': -1,   'C': 64},
    {'K': 12, 'D': 2, 'P': 768,  'C': 128},
    {'K': 16, 'D': 2, 'P': 384,  'C': 256},
    {'K': 16, 'D': 4, 'P': 128,  'C': 512},
]
_XDCONV_CFG = [
    {'K': 16, 'D': 4, 'pts_layer_idx': 3, 'qrs_layer_idx': 2},
    {'K': 16, 'D': 2, 'pts_layer_idx': 2, 'qrs_layer_idx': 1},
    {'K': 12, 'D': 2, 'pts_layer_idx': 1, 'qrs_layer_idx': 0},
]
_NFC = 3


def _elu(x):
    return jnp.where(x > 0, x, jnp.exp(x) - 1.0)


def _knn_body(KD1, P, rep_ref, ptst_ref, out_ref):
    rep = rep_ref[0]
    d2 = None
    for d in range(3):
        diff = rep[:, d:d + 1] - ptst_ref[0, d:d + 1, :]
        sq = diff * diff
        d2 = sq if d2 is None else d2 + sq
    Rb = rep.shape[0]
    iota = jax.lax.broadcasted_iota(jnp.int32, (Rb, P), 1).astype(jnp.float32)
    cur = d2
    idxf = None
    for i in range(KD1):
        if idxf is not None:
            cur = jnp.where(iota == idxf, jnp.float32(jnp.inf), cur)
        m = jnp.min(cur, axis=1, keepdims=True)
        idxf = jnp.min(jnp.where(cur == m, iota, jnp.float32(P)),
                       axis=1, keepdims=True)
        out_ref[0, :, i:i + 1] = idxf.astype(jnp.int32)


def _knn(rep, pts, KD1):
    B, R, _ = rep.shape
    P = pts.shape[1]
    ptst = jnp.transpose(pts, (0, 2, 1))
    Rb = 128
    return pl.pallas_call(
        functools.partial(_knn_body, KD1, P),
        grid=(B, R // Rb),
        in_specs=[
            pl.BlockSpec((1, Rb, 3), lambda b, r: (b, r, 0)),
            pl.BlockSpec((1, 3, P), lambda b, r: (b, 0, 0)),
        ],
        out_specs=pl.BlockSpec((1, Rb, KD1), lambda b, r: (b, r, 0)),
        out_shape=jax.ShapeDtypeStruct((B, R, KD1), jnp.int32),
    )(rep, ptst)


def _sc_gather(data2d, idx_flat):
    N = idx_flat.shape[0]
    C = data2d.shape[1]
    window = 512
    while window > 8 and (N % window != 0 or window * C * 4 > 131072):
        window //= 2
    mesh = plsc.VectorSubcoreMesh(core_axis_name="c", subcore_axis_name="s")
    idx2 = idx_flat.reshape(N // window, window)

    @functools.partial(pl.kernel,
                       out_type=jax.ShapeDtypeStruct((N, C), data2d.dtype),
                       mesh=mesh)
    def _gather_kernel(x_hbm, i_hbm, o_hbm):
        def body(i_vmem, o_vmem):
            pltpu.sync_copy(x_hbm.at[i_vmem.at[0]], o_vmem)

        pltpu.emit_pipeline(
            body,
            grid=(N // window,),
            in_specs=[pl.BlockSpec((1, window), lambda i: (i, 0))],
            out_specs=[pl.BlockSpec((window, C), lambda i: (i, 0))],
            core_axis_name=("c", "s"),
            dimension_semantics=(pltpu.PARALLEL,),
        )(i_hbm, o_hbm)

    return _gather_kernel(data2d, idx2)


def _mlp_body(nlayers, *refs):
    x_ref = refs[0]
    o_ref = refs[-1]
    h = x_ref[...]
    for i in range(nlayers):
        W = refs[1 + 2 * i][...]
        b = refs[2 + 2 * i][...]
        h = _elu(jnp.dot(h, W, preferred_element_type=jnp.float32) + b)
    o_ref[...] = h


def _mlp(x2d, layers):
    M = x2d.shape[0]
    Cout = layers[-1][0].shape[1]
    args = [x2d]
    for W, b in layers:
        args += [W, b.reshape(1, -1)]
    return pl.pallas_call(
        functools.partial(_mlp_body, len(layers)),
        out_shape=jax.ShapeDtypeStruct((M, Cout), jnp.float32),
    )(*args)


def _fold_body(dm, dw_ref, pwt_ref, o_ref):
    dwk = dw_ref[0]
    acc = None
    for m in range(dm):
        t = dwk[:, m:m + 1] * pwt_ref[m]
        acc = t if acc is None else acc + t
    o_ref[0] = acc


def _fold(dwaug, pw, Ctot, dm, Cout):
    K1 = dwaug.shape[0]
    pwt = pw.reshape(Ctot, dm, Cout).transpose(1, 0, 2)
    cb = Cout
    while dm * Ctot * cb * 4 > 16 * 1024 * 1024:
        cb //= 2
    return pl.pallas_call(
        functools.partial(_fold_body, dm),
        grid=(Cout // cb, K1),
        in_specs=[
            pl.BlockSpec((1, Ctot, dm), lambda ci, k: (k, 0, 0)),
            pl.BlockSpec((dm, Ctot, cb), lambda ci, k: (0, 0, ci)),
        ],
        out_specs=pl.BlockSpec((1, Ctot, cb), lambda ci, k: (k, 0, ci)),
        out_shape=jax.ShapeDtypeStruct((K1, Ctot, Cout), jnp.float32),
    )(dwaug, pwt)


def _xconv_body(K, Cmid, has_fts, dm1, *refs):
    rep_ref, pts_ref = refs[0], refs[1]
    refs = refs[2:]
    if has_fts:
        fts_ref, refs = refs[0], refs[1:]
    if dm1:
        (Wd1, bd1, Wd2, bd2, Wx0, bx0, Wx1, bx1, Wx2, bx2, dwr, dwb, pw,
         pwb, o_ref) = refs
    else:
        (Wd1, bd1, Wd2, bd2, Wx0, bx0, Wx1, bx1, Wx2, bx2, Waug, pwb,
         o_ref) = refs

    rep = rep_ref[0]
    ploc = [pts_ref[0, k] - rep for k in range(K)]

    f2 = []
    for k in range(K):
        acc = None
        for d in range(3):
            t = ploc[k][:, d:d + 1] * Wd1[d:d + 1, :]
            acc = t if acc is None else acc + t
        h = _elu(acc + bd1[...])
        h = _elu(jnp.dot(h, Wd2[...],
                               preferred_element_type=jnp.float32) + bd2[...])
        f2.append(h)

    Xacc = None
    for k in range(K):
        for d in range(3):
            t = ploc[k][:, d:d + 1] * Wx0[k * 3 + d:k * 3 + d + 1, :]
            Xacc = t if Xacc is None else Xacc + t
    X = _elu(Xacc + bx0[...])
    X = _elu(jnp.dot(X, Wx1[...],
                           preferred_element_type=jnp.float32) + bx1[...])
    X = jnp.dot(X, Wx2[...], preferred_element_type=jnp.float32) + bx2[...]

    if has_fts:
        fts = [fts_ref[0, j] for j in range(K)]

    if dm1:
        dw2f = None
        dw2r = None
        for k in range(K):
            accf = None
            accr = None
            for j in range(K):
                c = X[:, k * K + j:k * K + j + 1]
                t = c * f2[j]
                accf = t if accf is None else accf + t
                if has_fts:
                    t = c * fts[j]
                    accr = t if accr is None else accr + t
            tf = accf * dwr[k:k + 1, :Cmid]
            dw2f = tf if dw2f is None else dw2f + tf
            if has_fts:
                tr = accr * dwr[k:k + 1, Cmid:]
                dw2r = tr if dw2r is None else dw2r + tr
        out = jnp.dot(dw2f + dwb[:, :Cmid], pw[:Cmid, :],
                      preferred_element_type=jnp.float32)
        if has_fts:
            out = out + jnp.dot(dw2r + dwb[:, Cmid:], pw[Cmid:, :],
                                preferred_element_type=jnp.float32)
        o_ref[0] = _elu(out + pwb[...])
        return

    out = None
    for k in range(K):
        accf = None
        for j in range(K):
            c = X[:, k * K + j:k * K + j + 1]
            t = c * f2[j]
            accf = t if accf is None else accf + t
        term = jnp.dot(accf, Waug[k, :Cmid, :],
                       preferred_element_type=jnp.float32)
        if has_fts:
            accr = None
            for j in range(K):
                c = X[:, k * K + j:k * K + j + 1]
                t = c * fts[j]
                accr = t if accr is None else accr + t
            term = term + jnp.dot(accr, Waug[k, Cmid:, :],
                                  preferred_element_type=jnp.float32)
        out = term if out is None else out + term

    bias2 = jnp.sum(Waug[K], axis=0, keepdims=True) + pwb[...]
    o_ref[0] = _elu(out + bias2)


def _xconv(p, rep16, pts16, fts, K, Cmid, Cin, dm, Cout):
    B, R = rep16.shape[0], rep16.shape[1]
    Ctot = Cmid + Cin
    dm1 = dm == 1

    per_row = K * (Cin + Cmid + 16) * 4
    cap = 6 * 1024 * 1024
    Rb = R
    if R * per_row > cap:
        Rb = next((c for c in (512, 384, 256, 128)
                   if R % c == 0 and c * per_row <= cap), 128)
    args = [rep16, pts16]
    in_specs = [
        pl.BlockSpec((1, Rb, 16), lambda b, r, co: (b, r, 0)),
        pl.BlockSpec((1, K, Rb, 16), lambda b, r, co: (b, 0, r, 0)),
    ]
    if fts is not None:
        args.append(fts)
        in_specs.append(
            pl.BlockSpec((1, K, Rb, Cin), lambda b, r, co: (b, 0, r, 0)))
    KK = K * K
    wspecs = [
        (p['d1']['W'], (3, Cmid)), (p['d1']['b'].reshape(1, -1), (1, Cmid)),
        (p['d2']['W'], (Cmid, Cmid)), (p['d2']['b'].reshape(1, -1), (1, Cmid)),
        (p['x0']['W'], (3 * K, KK)), (p['x0']['b'].reshape(1, -1), (1, KK)),
        (p['x1']['W'], (KK, KK)), (p['x1']['b'].reshape(1, -1), (1, KK)),
        (p['x2']['W'], (KK, KK)), (p['x2']['b'].reshape(1, -1), (1, KK)),
    ]
    cob = Cout
    if not dm1:
        while (K + 1) * Ctot * cob * 4 > 6 * 1024 * 1024:
            cob //= 2
    for arr, shp in wspecs:
        args.append(arr)
        in_specs.append(
            pl.BlockSpec(shp, lambda b, r, co, _n=len(shp): (0,) * _n))
    if dm1:
        args += [p['dw'].reshape(K, Ctot), p['dwb'].reshape(1, Ctot),
                 p['pw'], p['pwb'].reshape(1, -1)]
        in_specs += [
            pl.BlockSpec((K, Ctot), lambda b, r, co: (0, 0)),
            pl.BlockSpec((1, Ctot), lambda b, r, co: (0, 0)),
            pl.BlockSpec((Ctot, Cout), lambda b, r, co: (0, 0)),
            pl.BlockSpec((1, Cout), lambda b, r, co: (0, 0)),
        ]
    else:
        dwaug = jnp.concatenate(
            [p['dw'], p['dwb'].reshape(1, Ctot, dm)], axis=0)
        Waug = _fold(dwaug, p['pw'], Ctot, dm, Cout)
        args += [Waug, p['pwb'].reshape(1, -1)]
        in_specs += [
            pl.BlockSpec((K + 1, Ctot, cob), lambda b, r, co: (0, 0, co)),
            pl.BlockSpec((1, cob), lambda b, r, co: (0, co)),
        ]
    return pl.pallas_call(
        functools.partial(_xconv_body, K, Cmid, fts is not None, dm1),
        grid=(B, R // Rb, Cout // cob),
        in_specs=in_specs,
        out_specs=pl.BlockSpec((1, Rb, cob), lambda b, r, co: (b, r, co)),
        out_shape=jax.ShapeDtypeStruct((B, R, Cout), jnp.float32),
    )(*args)


def _pad16(pts):
    B, P, _ = pts.shape
    return jnp.concatenate(
        [pts, jnp.zeros((B, P, 13), jnp.float32)], axis=-1)


def _gather_neighbors(data3d, nidx, keep=None):
    B, P, C = data3d.shape
    _, R, K = nidx.shape
    Cp = ((C + 127) // 128) * 128
    if Cp != C:
        data3d = jnp.concatenate(
            [data3d, jnp.zeros((B, P, Cp - C), jnp.float32)], axis=-1)
    flat = (nidx + (jnp.arange(B, dtype=jnp.int32) * P)[:, None, None])
    g = _sc_gather(data3d.reshape(B * P, Cp), flat.reshape(B * R * K))
    g = g.reshape(B, R, K, Cp)
    if keep is None:
        keep = C
    if keep != Cp:
        g = g[..., :keep]
    return g.transpose(0, 2, 1, 3)


def _nidx(rep, pts, K, D):
    idx = _knn(rep, pts, K * D + 1)
    return idx[:, :, 1::D][:, :, :K]


def kernel(x, params):
    rng = np.random.default_rng(0)
    B, NPTS, _ = x.shape
    C = [c['C'] for c in _XCONV_CFG]
    xc_meta = [
        (0, C[0] // 2, 4),
        (C[1] // 2, C[1] // 4, C[0] // 4),
        (C[2] // 2, C[2] // 4, C[1] // 4),
        (C[3] // 2, C[3] // 4, C[2] // 4),
    ]

    layer_pts = [x]
    outs = [None]
    prev = x
    prev_out = None
    for i, cfg in enumerate(_XCONV_CFG):
        if cfg['P'] != -1:
            sel = rng.choice(prev.shape[1], cfg['P'], replace=False)
            rep = prev[:, sel, :]
        else:
            rep = prev
        Cin, Cmid, dm = xc_meta[i]
        if i == 0:
            fts_full = None
        else:
            dp = params['dense%d' % i]
            fts_full = _mlp(prev_out.reshape(-1, prev_out.shape[-1]),
                            [(dp['W'], dp['b'])]).reshape(
                                prev.shape[0], prev.shape[1], -1)
        nidx = _nidx(rep, prev, cfg['K'], cfg['D'])
        pts16 = _gather_neighbors(prev, nidx, keep=16)
        ftsg = (None if fts_full is None
                else _gather_neighbors(fts_full, nidx))
        out = _xconv(params['xconv%d' % (i + 1)], _pad16(rep), pts16, ftsg,
                     cfg['K'], Cmid, Cin, dm, cfg['C'])
        layer_pts.append(rep)
        outs.append(out)
        prev = rep
        prev_out = out

    for i, cfg in enumerate(_XDCONV_CFG):
        this_out = outs[cfg['pts_layer_idx'] + 1] if i == 0 else outs[-1]
        rep = layer_pts[cfg['qrs_layer_idx'] + 1]
        rep2 = layer_pts[cfg['pts_layer_idx'] + 1]
        ci = this_out.shape[-1]
        co = C[cfg['qrs_layer_idx']]
        nidx = _nidx(rep, rep2, cfg['K'], cfg['D'])
        pts16 = _gather_neighbors(rep2, nidx, keep=16)
        this_r = _gather_neighbors(this_out, nidx)
        out = _xconv(params['deconv%d' % i], _pad16(rep), pts16, this_r,
                     cfg['K'], ci // 4, ci, 1, co)
        cat = jnp.concatenate([out, outs[cfg['qrs_layer_idx'] + 1]], axis=-1)
        dp = params['ddense%d' % i]
        out = _mlp(cat.reshape(-1, cat.shape[-1]),
                   [(dp['W'], dp['b'])]).reshape(B, -1, co)
        outs.append(out)

    h = outs[-1]
    fc_layers = [(params['fc%d' % i]['W'], params['fc%d' % i]['b'])
                 for i in range(_NFC)]
    out = _mlp(h.reshape(-1, h.shape[-1]), fc_layers)
    return out.reshape(B, NPTS, -1)


# speedup vs baseline: 4.9683x; 1.1422x over previous
import functools

import numpy as np
import jax
import jax.numpy as jnp
from jax.experimental import pallas as pl
from jax.experimental.pallas import tpu as pltpu
from jax.experimental.pallas import tpu_sc as plsc

_XCONV_CFG = [
    {'K': 8,  'D': 1, 'P': -1,   'C': 64},
    {'K': 12, 'D': 2, 'P': 768,  'C': 128},
    {'K': 16, 'D': 2, 'P': 384,  'C': 256},
    {'K': 16, 'D': 4, 'P': 128,  'C': 512},
]
_XDCONV_CFG = [
    {'K': 16, 'D': 4, 'pts_layer_idx': 3, 'qrs_layer_idx': 2},
    {'K': 16, 'D': 2, 'pts_layer_idx': 2, 'qrs_layer_idx': 1},
    {'K': 12, 'D': 2, 'pts_layer_idx': 1, 'qrs_layer_idx': 0},
]
_NFC = 3


def _elu(x):
    return jnp.where(x > 0, x, jnp.exp(x) - 1.0)


def _knn_body(KD1, P, rep_ref, ptst_ref, out_ref):
    rep = rep_ref[0]
    d2 = None
    for d in range(3):
        diff = rep[:, d:d + 1] - ptst_ref[0, d:d + 1, :]
        sq = diff * diff
        d2 = sq if d2 is None else d2 + sq
    Rb = rep.shape[0]
    iota = jax.lax.broadcasted_iota(jnp.int32, (Rb, P), 1).astype(jnp.float32)
    cur = d2
    idxf = None
    for i in range(KD1):
        if idxf is not None:
            cur = jnp.where(iota == idxf, jnp.float32(jnp.inf), cur)
        m = jnp.min(cur, axis=1, keepdims=True)
        idxf = jnp.min(jnp.where(cur == m, iota, jnp.float32(P)),
                       axis=1, keepdims=True)
        out_ref[0, :, i:i + 1] = idxf.astype(jnp.int32)


def _knn(rep, pts, KD1):
    B, R, _ = rep.shape
    P = pts.shape[1]
    ptst = jnp.transpose(pts, (0, 2, 1))
    Rb = 128
    return pl.pallas_call(
        functools.partial(_knn_body, KD1, P),
        grid=(B, R // Rb),
        in_specs=[
            pl.BlockSpec((1, Rb, 3), lambda b, r: (b, r, 0)),
            pl.BlockSpec((1, 3, P), lambda b, r: (b, 0, 0)),
        ],
        out_specs=pl.BlockSpec((1, Rb, KD1), lambda b, r: (b, r, 0)),
        out_shape=jax.ShapeDtypeStruct((B, R, KD1), jnp.int32),
    )(rep, ptst)


def _sc_gather(data2d, idx_flat):
    N = idx_flat.shape[0]
    C = data2d.shape[1]
    window = 512
    while window > 8 and (N % window != 0 or window * C * 4 > 131072):
        window //= 2
    mesh = plsc.VectorSubcoreMesh(core_axis_name="c", subcore_axis_name="s")
    idx2 = idx_flat.reshape(N // window, window)

    @functools.partial(pl.kernel,
                       out_type=jax.ShapeDtypeStruct((N, C), data2d.dtype),
                       mesh=mesh)
    def _gather_kernel(x_hbm, i_hbm, o_hbm):
        def body(i_vmem, o_vmem):
            pltpu.sync_copy(x_hbm.at[i_vmem.at[0]], o_vmem)

        pltpu.emit_pipeline(
            body,
            grid=(N // window,),
            in_specs=[pl.BlockSpec((1, window), lambda i: (i, 0))],
            out_specs=[pl.BlockSpec((window, C), lambda i: (i, 0))],
            core_axis_name=("c", "s"),
            dimension_semantics=(pltpu.PARALLEL,),
        )(i_hbm, o_hbm)

    return _gather_kernel(data2d, idx2)


def _mlp_body(nlayers, *refs):
    x_ref = refs[0]
    o_ref = refs[-1]
    h = x_ref[...]
    for i in range(nlayers):
        W = refs[1 + 2 * i][...]
        b = refs[2 + 2 * i][...]
        h = _elu(jnp.dot(h, W, preferred_element_type=jnp.float32) + b)
    o_ref[...] = h


def _mlp(x2d, layers):
    M = x2d.shape[0]
    Cout = layers[-1][0].shape[1]
    args = [x2d]
    for W, b in layers:
        args += [W, b.reshape(1, -1)]
    return pl.pallas_call(
        functools.partial(_mlp_body, len(layers)),
        out_shape=jax.ShapeDtypeStruct((M, Cout), jnp.float32),
    )(*args)


def _fold_body(dm, dw_ref, pwt_ref, o_ref):
    dwk = dw_ref[0]
    acc = None
    for m in range(dm):
        t = dwk[:, m:m + 1] * pwt_ref[m]
        acc = t if acc is None else acc + t
    o_ref[0] = acc


def _fold(dwaug, pw, Ctot, dm, Cout):
    K1 = dwaug.shape[0]
    pwt = pw.reshape(Ctot, dm, Cout).transpose(1, 0, 2)
    cb = Cout
    while dm * Ctot * cb * 4 > 16 * 1024 * 1024:
        cb //= 2
    return pl.pallas_call(
        functools.partial(_fold_body, dm),
        grid=(Cout // cb, K1),
        in_specs=[
            pl.BlockSpec((1, Ctot, dm), lambda ci, k: (k, 0, 0)),
            pl.BlockSpec((dm, Ctot, cb), lambda ci, k: (0, 0, ci)),
        ],
        out_specs=pl.BlockSpec((1, Ctot, cb), lambda ci, k: (k, 0, ci)),
        out_shape=jax.ShapeDtypeStruct((K1, Ctot, Cout), jnp.float32),
    )(dwaug, pwt)


def _xconv_body(K, Cmid, has_fts, dm1, *refs):
    rep_ref, pts_ref = refs[0], refs[1]
    refs = refs[2:]
    if has_fts:
        fts_ref, refs = refs[0], refs[1:]
    if dm1:
        (Wd1, bd1, Wd2, bd2, Wx0, bx0, Wx1, bx1, Wx2, bx2, dwr, dwb, pw,
         pwb, o_ref) = refs
    else:
        (Wd1, bd1, Wd2, bd2, Wx0, bx0, Wx1, bx1, Wx2, bx2, Waug, pwb,
         o_ref) = refs

    rep = rep_ref[0]
    ploc = [pts_ref[0, k] - rep for k in range(K)]

    f2 = []
    for k in range(K):
        acc = None
        for d in range(3):
            t = ploc[k][:, d:d + 1] * Wd1[d:d + 1, :]
            acc = t if acc is None else acc + t
        h = _elu(acc + bd1[...])
        h = _elu(jnp.dot(h, Wd2[...],
                               preferred_element_type=jnp.float32) + bd2[...])
        f2.append(h)

    Xacc = None
    for k in range(K):
        for d in range(3):
            t = ploc[k][:, d:d + 1] * Wx0[k * 3 + d:k * 3 + d + 1, :]
            Xacc = t if Xacc is None else Xacc + t
    X = _elu(Xacc + bx0[...])
    X = _elu(jnp.dot(X, Wx1[...],
                           preferred_element_type=jnp.float32) + bx1[...])
    X = jnp.dot(X, Wx2[...], preferred_element_type=jnp.float32) + bx2[...]

    if has_fts:
        fts = [fts_ref[0, j] for j in range(K)]

    if dm1:
        dw2f = None
        dw2r = None
        for k in range(K):
            accf = None
            accr = None
            for j in range(K):
                c = X[:, k * K + j:k * K + j + 1]
                t = c * f2[j]
                accf = t if accf is None else accf + t
                if has_fts:
                    t = c * fts[j]
                    accr = t if accr is None else accr + t
            tf = accf * dwr[k:k + 1, :Cmid]
            dw2f = tf if dw2f is None else dw2f + tf
            if has_fts:
                tr = accr * dwr[k:k + 1, Cmid:]
                dw2r = tr if dw2r is None else dw2r + tr
        out = jnp.dot(dw2f + dwb[:, :Cmid], pw[:Cmid, :],
                      preferred_element_type=jnp.float32)
        if has_fts:
            out = out + jnp.dot(dw2r + dwb[:, Cmid:], pw[Cmid:, :],
                                preferred_element_type=jnp.float32)
        o_ref[0] = _elu(out + pwb[...])
        return

    out = None
    for k in range(K):
        accf = None
        for j in range(K):
            c = X[:, k * K + j:k * K + j + 1]
            t = c * f2[j]
            accf = t if accf is None else accf + t
        term = jnp.dot(accf, Waug[k, :Cmid, :],
                       preferred_element_type=jnp.float32)
        if has_fts:
            accr = None
            for j in range(K):
                c = X[:, k * K + j:k * K + j + 1]
                t = c * fts[j]
                accr = t if accr is None else accr + t
            term = term + jnp.dot(accr, Waug[k, Cmid:, :],
                                  preferred_element_type=jnp.float32)
        out = term if out is None else out + term

    bias2 = jnp.sum(Waug[K], axis=0, keepdims=True) + pwb[...]
    o_ref[0] = _elu(out + bias2)


def _xconv(p, rep16, pts16, fts, K, Cmid, Cin, dm, Cout):
    B, R = rep16.shape[0], rep16.shape[1]
    Ctot = Cmid + Cin
    dm1 = dm == 1

    per_row = K * (Cin + Cmid + 16) * 4
    cap = 6 * 1024 * 1024
    Rb = R
    if R * per_row > cap:
        Rb = next((c for c in (512, 384, 256, 128)
                   if R % c == 0 and c * per_row <= cap), 128)
    args = [rep16, pts16]
    in_specs = [
        pl.BlockSpec((1, Rb, 16), lambda b, r, co: (b, r, 0)),
        pl.BlockSpec((1, K, Rb, 16), lambda b, r, co: (b, 0, r, 0)),
    ]
    if fts is not None:
        args.append(fts)
        in_specs.append(
            pl.BlockSpec((1, K, Rb, Cin), lambda b, r, co: (b, 0, r, 0)))
    KK = K * K
    wspecs = [
        (p['d1']['W'], (3, Cmid)), (p['d1']['b'].reshape(1, -1), (1, Cmid)),
        (p['d2']['W'], (Cmid, Cmid)), (p['d2']['b'].reshape(1, -1), (1, Cmid)),
        (p['x0']['W'], (3 * K, KK)), (p['x0']['b'].reshape(1, -1), (1, KK)),
        (p['x1']['W'], (KK, KK)), (p['x1']['b'].reshape(1, -1), (1, KK)),
        (p['x2']['W'], (KK, KK)), (p['x2']['b'].reshape(1, -1), (1, KK)),
    ]
    cob = Cout
    if not dm1:
        while (K + 1) * Ctot * cob * 4 > 6 * 1024 * 1024:
            cob //= 2
    for arr, shp in wspecs:
        args.append(arr)
        in_specs.append(
            pl.BlockSpec(shp, lambda b, r, co, _n=len(shp): (0,) * _n))
    if dm1:
        args += [p['dw'].reshape(K, Ctot), p['dwb'].reshape(1, Ctot),
                 p['pw'], p['pwb'].reshape(1, -1)]
        in_specs += [
            pl.BlockSpec((K, Ctot), lambda b, r, co: (0, 0)),
            pl.BlockSpec((1, Ctot), lambda b, r, co: (0, 0)),
            pl.BlockSpec((Ctot, Cout), lambda b, r, co: (0, 0)),
            pl.BlockSpec((1, Cout), lambda b, r, co: (0, 0)),
        ]
    else:
        dwaug = jnp.concatenate(
            [p['dw'], p['dwb'].reshape(1, Ctot, dm)], axis=0)
        Waug = _fold(dwaug, p['pw'], Ctot, dm, Cout)
        args += [Waug, p['pwb'].reshape(1, -1)]
        in_specs += [
            pl.BlockSpec((K + 1, Ctot, cob), lambda b, r, co: (0, 0, co)),
            pl.BlockSpec((1, cob), lambda b, r, co: (0, co)),
        ]
    return pl.pallas_call(
        functools.partial(_xconv_body, K, Cmid, fts is not None, dm1),
        grid=(B, R // Rb, Cout // cob),
        in_specs=in_specs,
        out_specs=pl.BlockSpec((1, Rb, cob), lambda b, r, co: (b, r, co)),
        out_shape=jax.ShapeDtypeStruct((B, R, Cout), jnp.float32),
    )(*args)


def _pad16(pts):
    B, P, _ = pts.shape
    return jnp.concatenate(
        [pts, jnp.zeros((B, P, 13), jnp.float32)], axis=-1)


def _gather_neighbors(data3d, nidx, keep=None):
    B, P, C = data3d.shape
    _, R, K = nidx.shape
    Cp = ((C + 127) // 128) * 128
    if Cp != C:
        data3d = jnp.concatenate(
            [data3d, jnp.zeros((B, P, Cp - C), jnp.float32)], axis=-1)
    flat = (nidx + (jnp.arange(B, dtype=jnp.int32) * P)[:, None, None])
    flat = flat.transpose(0, 2, 1)
    g = _sc_gather(data3d.reshape(B * P, Cp), flat.reshape(B * R * K))
    g = g.reshape(B, K, R, Cp)
    if keep is None:
        keep = C
    if keep != Cp:
        g = g[..., :keep]
    return g


def _nidx(rep, pts, K, D):
    idx = _knn(rep, pts, K * D + 1)
    return idx[:, :, 1::D][:, :, :K]


def kernel(x, params):
    rng = np.random.default_rng(0)
    B, NPTS, _ = x.shape
    C = [c['C'] for c in _XCONV_CFG]
    xc_meta = [
        (0, C[0] // 2, 4),
        (C[1] // 2, C[1] // 4, C[0] // 4),
        (C[2] // 2, C[2] // 4, C[1] // 4),
        (C[3] // 2, C[3] // 4, C[2] // 4),
    ]

    layer_pts = [x]
    outs = [None]
    prev = x
    prev_out = None
    for i, cfg in enumerate(_XCONV_CFG):
        if cfg['P'] != -1:
            sel = rng.choice(prev.shape[1], cfg['P'], replace=False)
            rep = prev[:, sel, :]
        else:
            rep = prev
        Cin, Cmid, dm = xc_meta[i]
        if i == 0:
            fts_full = None
        else:
            dp = params['dense%d' % i]
            fts_full = _mlp(prev_out.reshape(-1, prev_out.shape[-1]),
                            [(dp['W'], dp['b'])]).reshape(
                                prev.shape[0], prev.shape[1], -1)
        nidx = _nidx(rep, prev, cfg['K'], cfg['D'])
        pts16 = _gather_neighbors(prev, nidx, keep=16)
        ftsg = (None if fts_full is None
                else _gather_neighbors(fts_full, nidx))
        out = _xconv(params['xconv%d' % (i + 1)], _pad16(rep), pts16, ftsg,
                     cfg['K'], Cmid, Cin, dm, cfg['C'])
        layer_pts.append(rep)
        outs.append(out)
        prev = rep
        prev_out = out

    for i, cfg in enumerate(_XDCONV_CFG):
        this_out = outs[cfg['pts_layer_idx'] + 1] if i == 0 else outs[-1]
        rep = layer_pts[cfg['qrs_layer_idx'] + 1]
        rep2 = layer_pts[cfg['pts_layer_idx'] + 1]
        ci = this_out.shape[-1]
        co = C[cfg['qrs_layer_idx']]
        nidx = _nidx(rep, rep2, cfg['K'], cfg['D'])
        pts16 = _gather_neighbors(rep2, nidx, keep=16)
        this_r = _gather_neighbors(this_out, nidx)
        out = _xconv(params['deconv%d' % i], _pad16(rep), pts16, this_r,
                     cfg['K'], ci // 4, ci, 1, co)
        cat = jnp.concatenate([out, outs[cfg['qrs_layer_idx'] + 1]], axis=-1)
        dp = params['ddense%d' % i]
        out = _mlp(cat.reshape(-1, cat.shape[-1]),
                   [(dp['W'], dp['b'])]).reshape(B, -1, co)
        outs.append(out)

    h = outs[-1]
    fc_layers = [(params['fc%d' % i]['W'], params['fc%d' % i]['b'])
                 for i in range(_NFC)]
    out = _mlp(h.reshape(-1, h.shape[-1]), fc_layers)
    return out.reshape(B, NPTS, -1)


# speedup vs baseline: 5.0094x; 1.0083x over previous
import functools

import numpy as np
import jax
import jax.numpy as jnp
from jax.experimental import pallas as pl
from jax.experimental.pallas import tpu as pltpu
from jax.experimental.pallas import tpu_sc as plsc

_XCONV_CFG = [
    {'K': 8,  'D': 1, 'P': -1,   'C': 64},
    {'K': 12, 'D': 2, 'P': 768,  'C': 128},
    {'K': 16, 'D': 2, 'P': 384,  'C': 256},
    {'K': 16, 'D': 4, 'P': 128,  'C': 512},
]
_XDCONV_CFG = [
    {'K': 16, 'D': 4, 'pts_layer_idx': 3, 'qrs_layer_idx': 2},
    {'K': 16, 'D': 2, 'pts_layer_idx': 2, 'qrs_layer_idx': 1},
    {'K': 12, 'D': 2, 'pts_layer_idx': 1, 'qrs_layer_idx': 0},
]
_NFC = 3


def _elu(x):
    return jnp.where(x > 0, x, jnp.exp(x) - 1.0)


def _knn_body(KD1, P, rep_ref, ptst_ref, out_ref):
    rep = rep_ref[0]
    d2 = None
    for d in range(3):
        diff = rep[:, d:d + 1] - ptst_ref[0, d:d + 1, :]
        sq = diff * diff
        d2 = sq if d2 is None else d2 + sq
    Rb = rep.shape[0]
    iota = jax.lax.broadcasted_iota(jnp.int32, (Rb, P), 1).astype(jnp.float32)
    cur = d2
    idxf = None
    for i in range(KD1):
        if idxf is not None:
            cur = jnp.where(iota == idxf, jnp.float32(jnp.inf), cur)
        m = jnp.min(cur, axis=1, keepdims=True)
        idxf = jnp.min(jnp.where(cur == m, iota, jnp.float32(P)),
                       axis=1, keepdims=True)
        out_ref[0, :, i:i + 1] = idxf.astype(jnp.int32)


def _knn(rep, pts, KD1):
    B, R, _ = rep.shape
    P = pts.shape[1]
    ptst = jnp.transpose(pts, (0, 2, 1))
    Rb = 128
    return pl.pallas_call(
        functools.partial(_knn_body, KD1, P),
        grid=(B, R // Rb),
        in_specs=[
            pl.BlockSpec((1, Rb, 3), lambda b, r: (b, r, 0)),
            pl.BlockSpec((1, 3, P), lambda b, r: (b, 0, 0)),
        ],
        out_specs=pl.BlockSpec((1, Rb, KD1), lambda b, r: (b, r, 0)),
        out_shape=jax.ShapeDtypeStruct((B, R, KD1), jnp.int32),
    )(rep, ptst)


def _sc_gather(data2d, idx_flat):
    N = idx_flat.shape[0]
    C = data2d.shape[1]
    window = 512
    while window > 8 and (N % window != 0 or window * C * 4 > 131072):
        window //= 2
    mesh = plsc.VectorSubcoreMesh(core_axis_name="c", subcore_axis_name="s")
    idx2 = idx_flat.reshape(N // window, window)

    @functools.partial(pl.kernel,
                       out_type=jax.ShapeDtypeStruct((N, C), data2d.dtype),
                       mesh=mesh)
    def _gather_kernel(x_hbm, i_hbm, o_hbm):
        def body(i_vmem, o_vmem):
            pltpu.sync_copy(x_hbm.at[i_vmem.at[0]], o_vmem)

        pltpu.emit_pipeline(
            body,
            grid=(N // window,),
            in_specs=[pl.BlockSpec((1, window), lambda i: (i, 0))],
            out_specs=[pl.BlockSpec((window, C), lambda i: (i, 0))],
            core_axis_name=("c", "s"),
            dimension_semantics=(pltpu.PARALLEL,),
        )(i_hbm, o_hbm)

    return _gather_kernel(data2d, idx2)


def _mlp_body(nlayers, *refs):
    x_ref = refs[0]
    o_ref = refs[-1]
    h = x_ref[...]
    for i in range(nlayers):
        W = refs[1 + 2 * i][...]
        b = refs[2 + 2 * i][...]
        h = _elu(jnp.dot(h, W, preferred_element_type=jnp.float32) + b)
    o_ref[...] = h


def _mlp(x2d, layers):
    M = x2d.shape[0]
    Cout = layers[-1][0].shape[1]
    args = [x2d]
    for W, b in layers:
        args += [W, b.reshape(1, -1)]
    return pl.pallas_call(
        functools.partial(_mlp_body, len(layers)),
        out_shape=jax.ShapeDtypeStruct((M, Cout), jnp.float32),
    )(*args)


def _fold_body(dm, dw_ref, pwt_ref, o_ref):
    dwk = dw_ref[0]
    acc = None
    for m in range(dm):
        t = dwk[:, m:m + 1] * pwt_ref[m]
        acc = t if acc is None else acc + t
    o_ref[0] = acc


def _fold(dwaug, pw, Ctot, dm, Cout):
    K1 = dwaug.shape[0]
    pwt = pw.reshape(Ctot, dm, Cout).transpose(1, 0, 2)
    cb = Cout
    while dm * Ctot * cb * 4 > 16 * 1024 * 1024:
        cb //= 2
    return pl.pallas_call(
        functools.partial(_fold_body, dm),
        grid=(Cout // cb, K1),
        in_specs=[
            pl.BlockSpec((1, Ctot, dm), lambda ci, k: (k, 0, 0)),
            pl.BlockSpec((dm, Ctot, cb), lambda ci, k: (0, 0, ci)),
        ],
        out_specs=pl.BlockSpec((1, Ctot, cb), lambda ci, k: (k, 0, ci)),
        out_shape=jax.ShapeDtypeStruct((K1, Ctot, Cout), jnp.float32),
    )(dwaug, pwt)


def _xconv_body(K, Cmid, Cin, dm1, Cfp, *refs):
    rep_ref, nbr_ref = refs[0], refs[1]
    refs = refs[2:]
    if dm1:
        (Wd1, bd1, Wd2, bd2, Wx0, bx0, Wx1, bx1, Wx2, bx2, dwr, dwb, pw,
         pwb, o_ref) = refs
    else:
        (Wd1, bd1, Wd2, bd2, Wx0, bx0, Wx1, bx1, Wx2, bx2, Waug, pwb,
         o_ref) = refs

    has_fts = Cin > 0
    rep = rep_ref[0]
    if has_fts:
        nb = [nbr_ref[0, k] for k in range(K)]
        ploc = [n[:, Cfp:Cfp + 16] - rep for n in nb]
        fts = [n[:, :Cin] for n in nb]
    else:
        ploc = [nbr_ref[0, k] - rep for k in range(K)]

    f2 = []
    for k in range(K):
        acc = None
        for d in range(3):
            t = ploc[k][:, d:d + 1] * Wd1[d:d + 1, :]
            acc = t if acc is None else acc + t
        h = _elu(acc + bd1[...])
        h = _elu(jnp.dot(h, Wd2[...],
                               preferred_element_type=jnp.float32) + bd2[...])
        f2.append(h)

    Xacc = None
    for k in range(K):
        for d in range(3):
            t = ploc[k][:, d:d + 1] * Wx0[k * 3 + d:k * 3 + d + 1, :]
            Xacc = t if Xacc is None else Xacc + t
    X = _elu(Xacc + bx0[...])
    X = _elu(jnp.dot(X, Wx1[...],
                           preferred_element_type=jnp.float32) + bx1[...])
    X = jnp.dot(X, Wx2[...], preferred_element_type=jnp.float32) + bx2[...]

    if dm1:
        dw2f = None
        dw2r = None
        for k in range(K):
            accf = None
            accr = None
            for j in range(K):
                c = X[:, k * K + j:k * K + j + 1]
                t = c * f2[j]
                accf = t if accf is None else accf + t
                if has_fts:
                    t = c * fts[j]
                    accr = t if accr is None else accr + t
            tf = accf * dwr[k:k + 1, :Cmid]
            dw2f = tf if dw2f is None else dw2f + tf
            if has_fts:
                tr = accr * dwr[k:k + 1, Cmid:]
                dw2r = tr if dw2r is None else dw2r + tr
        out = jnp.dot(dw2f + dwb[:, :Cmid], pw[:Cmid, :],
                      preferred_element_type=jnp.float32)
        if has_fts:
            out = out + jnp.dot(dw2r + dwb[:, Cmid:], pw[Cmid:, :],
                                preferred_element_type=jnp.float32)
        o_ref[0] = _elu(out + pwb[...])
        return

    out = None
    for k in range(K):
        accf = None
        for j in range(K):
            c = X[:, k * K + j:k * K + j + 1]
            t = c * f2[j]
            accf = t if accf is None else accf + t
        term = jnp.dot(accf, Waug[k, :Cmid, :],
                       preferred_element_type=jnp.float32)
        if has_fts:
            accr = None
            for j in range(K):
                c = X[:, k * K + j:k * K + j + 1]
                t = c * fts[j]
                accr = t if accr is None else accr + t
            term = term + jnp.dot(accr, Waug[k, Cmid:, :],
                                  preferred_element_type=jnp.float32)
        out = term if out is None else out + term

    bias2 = jnp.sum(Waug[K], axis=0, keepdims=True) + pwb[...]
    o_ref[0] = _elu(out + bias2)


def _xconv(p, rep16, nbr, K, Cmid, Cin, dm, Cout):
    B, R = rep16.shape[0], rep16.shape[1]
    Ctot = Cmid + Cin
    dm1 = dm == 1
    if Cin > 0:
        Cfp = ((Cin + 127) // 128) * 128
        Wn = Cfp + 128
    else:
        Cfp = 0
        Wn = 16

    per_row = K * (Wn + Cmid + 16) * 4
    cap = 6 * 1024 * 1024
    Rb = R
    if R * per_row > cap:
        Rb = next((c for c in (512, 384, 256, 128)
                   if R % c == 0 and c * per_row <= cap), 128)
    args = [rep16, nbr]
    in_specs = [
        pl.BlockSpec((1, Rb, 16), lambda b, r, co: (b, r, 0)),
        pl.BlockSpec((1, K, Rb, Wn), lambda b, r, co: (b, 0, r, 0)),
    ]
    KK = K * K
    wspecs = [
        (p['d1']['W'], (3, Cmid)), (p['d1']['b'].reshape(1, -1), (1, Cmid)),
        (p['d2']['W'], (Cmid, Cmid)), (p['d2']['b'].reshape(1, -1), (1, Cmid)),
        (p['x0']['W'], (3 * K, KK)), (p['x0']['b'].reshape(1, -1), (1, KK)),
        (p['x1']['W'], (KK, KK)), (p['x1']['b'].reshape(1, -1), (1, KK)),
        (p['x2']['W'], (KK, KK)), (p['x2']['b'].reshape(1, -1), (1, KK)),
    ]
    cob = Cout
    if not dm1:
        while (K + 1) * Ctot * cob * 4 > 6 * 1024 * 1024:
            cob //= 2
    for arr, shp in wspecs:
        args.append(arr)
        in_specs.append(
            pl.BlockSpec(shp, lambda b, r, co, _n=len(shp): (0,) * _n))
    if dm1:
        args += [p['dw'].reshape(K, Ctot), p['dwb'].reshape(1, Ctot),
                 p['pw'], p['pwb'].reshape(1, -1)]
        in_specs += [
            pl.BlockSpec((K, Ctot), lambda b, r, co: (0, 0)),
            pl.BlockSpec((1, Ctot), lambda b, r, co: (0, 0)),
            pl.BlockSpec((Ctot, Cout), lambda b, r, co: (0, 0)),
            pl.BlockSpec((1, Cout), lambda b, r, co: (0, 0)),
        ]
    else:
        dwaug = jnp.concatenate(
            [p['dw'], p['dwb'].reshape(1, Ctot, dm)], axis=0)
        Waug = _fold(dwaug, p['pw'], Ctot, dm, Cout)
        args += [Waug, p['pwb'].reshape(1, -1)]
        in_specs += [
            pl.BlockSpec((K + 1, Ctot, cob), lambda b, r, co: (0, 0, co)),
            pl.BlockSpec((1, cob), lambda b, r, co: (0, co)),
        ]
    return pl.pallas_call(
        functools.partial(_xconv_body, K, Cmid, Cin, dm1, Cfp),
        grid=(B, R // Rb, Cout // cob),
        in_specs=in_specs,
        out_specs=pl.BlockSpec((1, Rb, cob), lambda b, r, co: (b, r, co)),
        out_shape=jax.ShapeDtypeStruct((B, R, Cout), jnp.float32),
    )(*args)


def _pad16(pts):
    B, P, _ = pts.shape
    return jnp.concatenate(
        [pts, jnp.zeros((B, P, 13), jnp.float32)], axis=-1)


def _gather_neighbors(data3d, nidx, keep=None):
    B, P, C = data3d.shape
    _, R, K = nidx.shape
    Cp = ((C + 127) // 128) * 128
    if Cp != C:
        data3d = jnp.concatenate(
            [data3d, jnp.zeros((B, P, Cp - C), jnp.float32)], axis=-1)
    flat = (nidx + (jnp.arange(B, dtype=jnp.int32) * P)[:, None, None])
    flat = flat.transpose(0, 2, 1)
    g = _sc_gather(data3d.reshape(B * P, Cp), flat.reshape(B * R * K))
    g = g.reshape(B, K, R, Cp)
    if keep is None:
        keep = C
    if keep != Cp:
        g = g[..., :keep]
    return g


def _gather_fused(pts, fts, nidx):
    B, P, C = fts.shape
    _, R, K = nidx.shape
    Cfp = ((C + 127) // 128) * 128
    parts = [fts]
    if Cfp != C:
        parts.append(jnp.zeros((B, P, Cfp - C), jnp.float32))
    parts += [pts, jnp.zeros((B, P, 125), jnp.float32)]
    data = jnp.concatenate(parts, axis=-1)
    flat = (nidx + (jnp.arange(B, dtype=jnp.int32) * P)[:, None, None])
    flat = flat.transpose(0, 2, 1)
    g = _sc_gather(data.reshape(B * P, Cfp + 128), flat.reshape(B * R * K))
    return g.reshape(B, K, R, Cfp + 128)


def _nidx(rep, pts, K, D):
    idx = _knn(rep, pts, K * D + 1)
    return idx[:, :, 1::D][:, :, :K]


def kernel(x, params):
    rng = np.random.default_rng(0)
    B, NPTS, _ = x.shape
    C = [c['C'] for c in _XCONV_CFG]
    xc_meta = [
        (0, C[0] // 2, 4),
        (C[1] // 2, C[1] // 4, C[0] // 4),
        (C[2] // 2, C[2] // 4, C[1] // 4),
        (C[3] // 2, C[3] // 4, C[2] // 4),
    ]

    layer_pts = [x]
    outs = [None]
    prev = x
    prev_out = None
    for i, cfg in enumerate(_XCONV_CFG):
        if cfg['P'] != -1:
            sel = rng.choice(prev.shape[1], cfg['P'], replace=False)
            rep = prev[:, sel, :]
        else:
            rep = prev
        Cin, Cmid, dm = xc_meta[i]
        if i == 0:
            fts_full = None
        else:
            dp = params['dense%d' % i]
            fts_full = _mlp(prev_out.reshape(-1, prev_out.shape[-1]),
                            [(dp['W'], dp['b'])]).reshape(
                                prev.shape[0], prev.shape[1], -1)
        nidx = _nidx(rep, prev, cfg['K'], cfg['D'])
        if fts_full is None:
            nbr = _gather_neighbors(prev, nidx, keep=16)
        else:
            nbr = _gather_fused(prev, fts_full, nidx)
        out = _xconv(params['xconv%d' % (i + 1)], _pad16(rep), nbr,
                     cfg['K'], Cmid, Cin, dm, cfg['C'])
        layer_pts.append(rep)
        outs.append(out)
        prev = rep
        prev_out = out

    for i, cfg in enumerate(_XDCONV_CFG):
        this_out = outs[cfg['pts_layer_idx'] + 1] if i == 0 else outs[-1]
        rep = layer_pts[cfg['qrs_layer_idx'] + 1]
        rep2 = layer_pts[cfg['pts_layer_idx'] + 1]
        ci = this_out.shape[-1]
        co = C[cfg['qrs_layer_idx']]
        nidx = _nidx(rep, rep2, cfg['K'], cfg['D'])
        nbr = _gather_fused(rep2, this_out, nidx)
        out = _xconv(params['deconv%d' % i], _pad16(rep), nbr,
                     cfg['K'], ci // 4, ci, 1, co)
        cat = jnp.concatenate([out, outs[cfg['qrs_layer_idx'] + 1]], axis=-1)
        dp = params['ddense%d' % i]
        out = _mlp(cat.reshape(-1, cat.shape[-1]),
                   [(dp['W'], dp['b'])]).reshape(B, -1, co)
        outs.append(out)

    h = outs[-1]
    fc_layers = [(params['fc%d' % i]['W'], params['fc%d' % i]['b'])
                 for i in range(_NFC)]
    out = _mlp(h.reshape(-1, h.shape[-1]), fc_layers)
    return out.reshape(B, NPTS, -1)
